# baseline (speedup 1.0000x reference)
import functools

import jax
import jax.numpy as jnp
from jax import lax
from jax.experimental import pallas as pl
from jax.experimental.pallas import tpu as pltpu
from jax.experimental.pallas import tpu_sc as plsc

N = 10000
E = 320000
H = 128
G = 512
T_EDGE = 4
NC = 2
NS = 16
NW = NC * NS
C = 64
EPT = E // NW
PT = 10240
CHUNKS = PT // C
NPAD = 10240
STRIPE = NPAD // NS
SCPT = STRIPE // C
BN = 1000


def _featurize_body(nt_ref, emb_ref, rel_ref, h_ref, h4_ref):
    col = lax.broadcasted_iota(jnp.int32, (BN, 128), 1)
    onehot = (nt_ref[...] == col).astype(jnp.float32)
    h = jnp.dot(onehot, emb_ref[...], preferred_element_type=jnp.float32)
    h_ref[...] = h
    for t in range(T_EDGE):
        h4_ref[t] = h * rel_ref[t]


def _featurize(nt2d, emb_pad, rel0):
    return pl.pallas_call(
        _featurize_body,
        grid=(N // BN,),
        in_specs=[
            pl.BlockSpec((BN, 1), lambda i: (i, 0)),
            pl.BlockSpec((128, H), lambda i: (0, 0)),
            pl.BlockSpec((T_EDGE, H), lambda i: (0, 0)),
        ],
        out_specs=[
            pl.BlockSpec((BN, H), lambda i: (i, 0)),
            pl.BlockSpec((T_EDGE, BN, H), lambda i: (0, i, 0)),
        ],
        out_shape=[
            jax.ShapeDtypeStruct((N, H), jnp.float32),
            jax.ShapeDtypeStruct((T_EDGE, N, H), jnp.float32),
        ],
    )(nt2d, emb_pad, rel0)


DEPTH = 4
UNROLL = 2
PAIRS = CHUNKS // (DEPTH * UNROLL)


def _make_edge_pass(with_deg):
    mesh = plsc.VectorSubcoreMesh(core_axis_name="c", subcore_axis_name="s",
                                  num_cores=NC, num_subcores=NS)
    out_type = [jax.ShapeDtypeStruct((NC, NPAD, H), jnp.float32)]
    if with_deg:
        out_type.append(jax.ShapeDtypeStruct((NC, NPAD), jnp.float32))
    NIB = DEPTH * UNROLL
    scratch = (
        [pltpu.VMEM((2, C), jnp.int32) for _ in range(NIB)]
        + [pltpu.VMEM((C, H), jnp.float32) for _ in range(DEPTH)]
        + [
            pltpu.VMEM((C,), jnp.float32),
            pltpu.VMEM_SHARED((NPAD, H), jnp.float32),
            pltpu.VMEM_SHARED((NPAD,), jnp.float32),
        ]
        + [pltpu.SemaphoreType.DMA for _ in range(NIB + 2 * DEPTH + 2)]
    )

    def body(h4_hbm, idx_hbm, zeros_hbm, zeros1_hbm, ones_hbm, *rest):
        if with_deg:
            p_hbm, deg_hbm = rest[0], rest[1]
            scr = rest[2:]
        else:
            p_hbm = rest[0]
            deg_hbm = None
            scr = rest[1:]
        ib = scr[0:NIB]
        gb = scr[NIB:NIB + DEPTH]
        ones_v, agg_sh, deg_sh = scr[NIB + DEPTH:NIB + DEPTH + 3]
        isem = scr[NIB + DEPTH + 3:NIB + DEPTH + 3 + NIB]
        gsem = scr[NIB + DEPTH + 3 + NIB:NIB + DEPTH + 3 + NIB + DEPTH]
        ssem = scr[NIB + DEPTH + 3 + NIB + DEPTH:
                   NIB + DEPTH + 3 + NIB + 2 * DEPTH]
        dsem = scr[-2]
        zsem = scr[-1]
        cid = lax.axis_index("c")
        sid = lax.axis_index("s")
        w = cid * NS + sid
        base = sid * STRIPE
        stripe_sl = pl.ds(base, STRIPE)

        zd = [pltpu.async_copy(zeros_hbm, agg_sh.at[pl.ds(base + k * C, C)],
                               zsem) for k in range(SCPT)]
        if with_deg:
            zd.append(pltpu.async_copy(zeros1_hbm, deg_sh.at[stripe_sl],
                                       zsem))
        pltpu.sync_copy(ones_hbm, ones_v)
        for d in zd:
            d.wait()
        plsc.subcore_barrier()

        def pair_body(p, carry):
            c0 = p * NIB
            idesc = [pltpu.async_copy(idx_hbm.at[w, c0 + m], ib[m], isem[m])
                     for m in range(NIB)]
            for u in range(UNROLL):
                gd = []
                for j in range(DEPTH):
                    m = u * DEPTH + j
                    idesc[m].wait()
                    gd.append(pltpu.async_copy(h4_hbm.at[ib[m].at[0]],
                                               gb[j], gsem[j]))
                sd = []
                dd = []
                for j in range(DEPTH):
                    m = u * DEPTH + j
                    gd[j].wait()
                    sd.append(pltpu.async_copy(gb[j],
                                               agg_sh.at[ib[m].at[1]],
                                               ssem[j], add=True))
                    if with_deg:
                        dd.append(pltpu.async_copy(ones_v,
                                                   deg_sh.at[ib[m].at[1]],
                                                   dsem, add=True))
                for d in sd:
                    d.wait()
                for d in dd:
                    d.wait()
            return carry

        lax.fori_loop(0, PAIRS, pair_body, 0)
        plsc.subcore_barrier()

        rd = [pltpu.async_copy(agg_sh.at[pl.ds(base + k * C, C)],
                               p_hbm.at[cid, pl.ds(base + k * C, C)],
                               zsem) for k in range(SCPT)]
        if with_deg:
            rd.append(pltpu.async_copy(deg_sh.at[stripe_sl],
                                       deg_hbm.at[cid, stripe_sl], zsem))
        for d in rd:
            d.wait()

    return functools.partial(
        pl.kernel, body, out_type=out_type, mesh=mesh, scratch_types=scratch
    )


@functools.lru_cache(maxsize=None)
def _edge_pass_deg():
    return _make_edge_pass(True)()


@functools.lru_cache(maxsize=None)
def _edge_pass():
    return _make_edge_pass(False)()


def _layer_body(p_ref, degp_ref, h_ref, w_ref, ws_ref, rel_ref,
                h1_ref, h4_ref, invd_ref):
    deg = jnp.maximum(degp_ref[0] + degp_ref[1], 1.0)
    invd = 1.0 / deg
    agg = (p_ref[0] + p_ref[1]) * invd
    h1 = jnp.maximum(
        jnp.dot(agg, w_ref[...], preferred_element_type=jnp.float32)
        + jnp.dot(h_ref[...], ws_ref[...], preferred_element_type=jnp.float32),
        0.0)
    h1_ref[...] = h1
    invd_ref[...] = invd
    for t in range(T_EDGE):
        h4_ref[t] = h1 * rel_ref[t]


def _layer_update(p, degp3, h, w, ws, rel1):
    return pl.pallas_call(
        _layer_body,
        grid=(N // BN,),
        in_specs=[
            pl.BlockSpec((NC, BN, H), lambda i: (0, i, 0)),
            pl.BlockSpec((NC, BN, 1), lambda i: (0, i, 0)),
            pl.BlockSpec((BN, H), lambda i: (i, 0)),
            pl.BlockSpec((H, H), lambda i: (0, 0)),
            pl.BlockSpec((H, H), lambda i: (0, 0)),
            pl.BlockSpec((T_EDGE, H), lambda i: (0, 0)),
        ],
        out_specs=[
            pl.BlockSpec((BN, H), lambda i: (i, 0)),
            pl.BlockSpec((T_EDGE, BN, H), lambda i: (0, i, 0)),
            pl.BlockSpec((BN, 1), lambda i: (i, 0)),
        ],
        out_shape=[
            jax.ShapeDtypeStruct((N, H), jnp.float32),
            jax.ShapeDtypeStruct((T_EDGE, N, H), jnp.float32),
            jax.ShapeDtypeStruct((N, 1), jnp.float32),
        ],
    )(p, degp3, h, w, ws, rel1)


def _final_body(p_ref, invd_ref, h_ref, w_ref, ws_ref, gid_ref,
                out_ref, acc_ref, cnt_ref):
    i = pl.program_id(0)
    agg = (p_ref[0] + p_ref[1]) * invd_ref[...]
    h2 = jnp.maximum(
        jnp.dot(agg, w_ref[...], preferred_element_type=jnp.float32)
        + jnp.dot(h_ref[...], ws_ref[...], preferred_element_type=jnp.float32),
        0.0)
    col = lax.broadcasted_iota(jnp.int32, (BN, G), 1)
    onehot = (gid_ref[...] == col).astype(jnp.float32)
    psum = lax.dot_general(onehot, h2, (((0,), (0,)), ((), ())),
                           preferred_element_type=jnp.float32)
    ones_col = jnp.ones((BN, 1), jnp.float32)
    csum = lax.dot_general(onehot, ones_col, (((0,), (0,)), ((), ())),
                           preferred_element_type=jnp.float32)

    @pl.when(i == 0)
    def _():
        acc_ref[...] = jnp.zeros_like(acc_ref)
        cnt_ref[...] = jnp.zeros_like(cnt_ref)

    acc_ref[...] += psum
    cnt_ref[...] += csum

    @pl.when(i == pl.num_programs(0) - 1)
    def _():
        out_ref[...] = acc_ref[...] / jnp.maximum(cnt_ref[...], 1.0)


def _final(p, invd, h, w, ws, gid2d):
    return pl.pallas_call(
        _final_body,
        grid=(N // BN,),
        in_specs=[
            pl.BlockSpec((NC, BN, H), lambda i: (0, i, 0)),
            pl.BlockSpec((BN, 1), lambda i: (i, 0)),
            pl.BlockSpec((BN, H), lambda i: (i, 0)),
            pl.BlockSpec((H, H), lambda i: (0, 0)),
            pl.BlockSpec((H, H), lambda i: (0, 0)),
            pl.BlockSpec((BN, 1), lambda i: (i, 0)),
        ],
        out_specs=pl.BlockSpec((G, H), lambda i: (0, 0)),
        out_shape=jax.ShapeDtypeStruct((G, H), jnp.float32),
        scratch_shapes=[
            pltpu.VMEM((G, H), jnp.float32),
            pltpu.VMEM((G, 1), jnp.float32),
        ],
    )(p, invd, h, w, ws, gid2d)


def kernel(node_types, edge_index, edge_types, graph_ids, node_emb, rel_emb,
           W, W_self):
    node_types = node_types.astype(jnp.int32)
    src = edge_index[0].astype(jnp.int32).reshape(NW, EPT)
    dst = edge_index[1].astype(jnp.int32).reshape(NW, EPT)
    typ = edge_types.astype(jnp.int32).reshape(NW, EPT)

    pad = PT - EPT
    pad_g = jnp.broadcast_to((jnp.arange(pad, dtype=jnp.int32) * 131) % N,
                             (NW, pad))
    pad_dst = jnp.broadcast_to(N + jnp.arange(pad, dtype=jnp.int32), (NW, pad))
    idx_all = jnp.stack([
        jnp.concatenate([typ * N + src, pad_g], axis=1).reshape(NW, CHUNKS, C),
        jnp.concatenate([dst, pad_dst], axis=1).reshape(NW, CHUNKS, C),
    ], axis=2)

    zeros = jnp.zeros((C, H), jnp.float32)
    zeros1 = jnp.zeros((STRIPE,), jnp.float32)
    ones = jnp.ones((C,), jnp.float32)
    emb_pad = jnp.pad(node_emb, ((0, 128 - node_emb.shape[0]), (0, 0)))

    h0, h4_0 = _featurize(node_types.reshape(N, 1), emb_pad, rel_emb[0])
    p0, degp = _edge_pass_deg()(h4_0.reshape(T_EDGE * N, H), idx_all,
                                zeros, zeros1, ones)
    h1, h4_1, invd = _layer_update(p0, degp.reshape(NC, NPAD, 1), h0,
                                   W[0], W_self[0], rel_emb[1])
    (p1,) = _edge_pass()(h4_1.reshape(T_EDGE * N, H), idx_all, zeros, zeros1,
                         ones)
    pooled = _final(p1, invd, h1, W[1], W_self[1],
                    graph_ids.astype(jnp.int32).reshape(N, 1))
    return pooled


# speedup vs baseline: 1.0891x; 1.0891x over previous
import functools

import jax
import jax.numpy as jnp
from jax import lax
from jax.experimental import pallas as pl
from jax.experimental.pallas import tpu as pltpu
from jax.experimental.pallas import tpu_sc as plsc

N = 10000
E = 320000
H = 128
G = 512
T_EDGE = 4
NC = 2
NS = 16
NW = NC * NS
C = 64
EPT = E // NW
PT = 10240
CHUNKS = PT // C
NPAD = 10240
STRIPE = NPAD // NS
SCPT = STRIPE // C
BN = 2000


def _featurize_body(nt_ref, emb_ref, rel_ref, h_ref, h4_ref):
    col = lax.broadcasted_iota(jnp.int32, (BN, 128), 1)
    onehot = (nt_ref[...] == col).astype(jnp.float32)
    h = jnp.dot(onehot, emb_ref[...], preferred_element_type=jnp.float32)
    h_ref[...] = h
    for t in range(T_EDGE):
        h4_ref[t] = h * rel_ref[t]


def _featurize(nt2d, emb_pad, rel0):
    return pl.pallas_call(
        _featurize_body,
        grid=(N // BN,),
        in_specs=[
            pl.BlockSpec((BN, 1), lambda i: (i, 0)),
            pl.BlockSpec((128, H), lambda i: (0, 0)),
            pl.BlockSpec((T_EDGE, H), lambda i: (0, 0)),
        ],
        out_specs=[
            pl.BlockSpec((BN, H), lambda i: (i, 0)),
            pl.BlockSpec((T_EDGE, BN, H), lambda i: (0, i, 0)),
        ],
        out_shape=[
            jax.ShapeDtypeStruct((N, H), jnp.float32),
            jax.ShapeDtypeStruct((T_EDGE, N, H), jnp.float32),
        ],
    )(nt2d, emb_pad, rel0)


DEPTH = 4
UNROLL = 2
PAIRS = CHUNKS // (DEPTH * UNROLL)


def _make_edge_pass(with_deg):
    mesh = plsc.VectorSubcoreMesh(core_axis_name="c", subcore_axis_name="s",
                                  num_cores=NC, num_subcores=NS)
    out_type = [jax.ShapeDtypeStruct((NC, NPAD, H), jnp.float32)]
    if with_deg:
        out_type.append(jax.ShapeDtypeStruct((NC, NPAD), jnp.float32))
    NIB = DEPTH * UNROLL
    scratch = (
        [pltpu.VMEM((2, C), jnp.int32) for _ in range(NIB)]
        + [pltpu.VMEM((C, H), jnp.float32) for _ in range(DEPTH)]
        + [
            pltpu.VMEM((C,), jnp.float32),
            pltpu.VMEM_SHARED((NPAD, H), jnp.float32),
            pltpu.VMEM_SHARED((NPAD,), jnp.float32),
        ]
        + [pltpu.SemaphoreType.DMA for _ in range(NIB + 2 * DEPTH + 1)]
    )

    def body(h4_hbm, idx_hbm, zeros_hbm, ones_hbm, *rest):
        if with_deg:
            p_hbm, deg_hbm = rest[0], rest[1]
            scr = rest[2:]
        else:
            p_hbm = rest[0]
            deg_hbm = None
            scr = rest[1:]
        ib = scr[0:NIB]
        gb = scr[NIB:NIB + DEPTH]
        ones_v, agg_sh, deg_sh = scr[NIB + DEPTH:NIB + DEPTH + 3]
        isem = scr[NIB + DEPTH + 3:NIB + DEPTH + 3 + NIB]
        gsem = scr[NIB + DEPTH + 3 + NIB:NIB + DEPTH + 3 + NIB + DEPTH]
        ssem = scr[NIB + DEPTH + 3 + NIB + DEPTH:
                   NIB + DEPTH + 3 + NIB + 2 * DEPTH]
        dsem = scr[-1]
        cid = lax.axis_index("c")
        sid = lax.axis_index("s")
        w = cid * NS + sid
        base = sid * STRIPE

        pltpu.sync_copy(zeros_hbm, gb[0])
        pltpu.sync_copy(ones_hbm, ones_v)
        for k in range(SCPT):
            sl = pl.ds(base + k * C, C)
            pltpu.sync_copy(gb[0], agg_sh.at[sl])
            if with_deg:
                pltpu.sync_copy(gb[0].at[0, pl.ds(0, C)], deg_sh.at[sl])
        plsc.subcore_barrier()

        def pair_body(p, carry):
            c0 = p * NIB
            idesc = [pltpu.async_copy(idx_hbm.at[w, c0 + m], ib[m], isem[m])
                     for m in range(NIB)]
            for u in range(UNROLL):
                gd = []
                for j in range(DEPTH):
                    m = u * DEPTH + j
                    idesc[m].wait()
                    gd.append(pltpu.async_copy(h4_hbm.at[ib[m].at[0]],
                                               gb[j], gsem[j]))
                sd = []
                dd = []
                for j in range(DEPTH):
                    m = u * DEPTH + j
                    gd[j].wait()
                    sd.append(pltpu.async_copy(gb[j],
                                               agg_sh.at[ib[m].at[1]],
                                               ssem[j], add=True))
                    if with_deg:
                        dd.append(pltpu.async_copy(ones_v,
                                                   deg_sh.at[ib[m].at[1]],
                                                   dsem, add=True))
                for d in sd:
                    d.wait()
                for d in dd:
                    d.wait()
            return carry

        lax.fori_loop(0, PAIRS, pair_body, 0)
        plsc.subcore_barrier()

        for k in range(SCPT):
            sl = pl.ds(base + k * C, C)
            pltpu.sync_copy(agg_sh.at[sl], gb[0])
            pltpu.sync_copy(gb[0], p_hbm.at[cid, sl])
            if with_deg:
                pltpu.sync_copy(deg_sh.at[sl], ones_v)
                pltpu.sync_copy(ones_v, deg_hbm.at[cid, sl])

    return functools.partial(
        pl.kernel, body, out_type=out_type, mesh=mesh, scratch_types=scratch
    )


@functools.lru_cache(maxsize=None)
def _edge_pass_deg():
    return _make_edge_pass(True)()


@functools.lru_cache(maxsize=None)
def _edge_pass():
    return _make_edge_pass(False)()


def _layer_body(p_ref, degp_ref, h_ref, w_ref, ws_ref, rel_ref,
                h1_ref, h4_ref, invd_ref):
    deg = jnp.maximum(degp_ref[0] + degp_ref[1], 1.0)
    invd = 1.0 / deg
    agg = (p_ref[0] + p_ref[1]) * invd
    h1 = jnp.maximum(
        jnp.dot(agg, w_ref[...], preferred_element_type=jnp.float32)
        + jnp.dot(h_ref[...], ws_ref[...], preferred_element_type=jnp.float32),
        0.0)
    h1_ref[...] = h1
    invd_ref[...] = invd
    for t in range(T_EDGE):
        h4_ref[t] = h1 * rel_ref[t]


def _layer_update(p, degp3, h, w, ws, rel1):
    return pl.pallas_call(
        _layer_body,
        grid=(N // BN,),
        in_specs=[
            pl.BlockSpec((NC, BN, H), lambda i: (0, i, 0)),
            pl.BlockSpec((NC, BN, 1), lambda i: (0, i, 0)),
            pl.BlockSpec((BN, H), lambda i: (i, 0)),
            pl.BlockSpec((H, H), lambda i: (0, 0)),
            pl.BlockSpec((H, H), lambda i: (0, 0)),
            pl.BlockSpec((T_EDGE, H), lambda i: (0, 0)),
        ],
        out_specs=[
            pl.BlockSpec((BN, H), lambda i: (i, 0)),
            pl.BlockSpec((T_EDGE, BN, H), lambda i: (0, i, 0)),
            pl.BlockSpec((BN, 1), lambda i: (i, 0)),
        ],
        out_shape=[
            jax.ShapeDtypeStruct((N, H), jnp.float32),
            jax.ShapeDtypeStruct((T_EDGE, N, H), jnp.float32),
            jax.ShapeDtypeStruct((N, 1), jnp.float32),
        ],
    )(p, degp3, h, w, ws, rel1)


def _final_body(p_ref, invd_ref, h_ref, w_ref, ws_ref, gid_ref,
                out_ref, acc_ref, cnt_ref):
    i = pl.program_id(0)
    agg = (p_ref[0] + p_ref[1]) * invd_ref[...]
    h2 = jnp.maximum(
        jnp.dot(agg, w_ref[...], preferred_element_type=jnp.float32)
        + jnp.dot(h_ref[...], ws_ref[...], preferred_element_type=jnp.float32),
        0.0)
    col = lax.broadcasted_iota(jnp.int32, (BN, G), 1)
    onehot = (gid_ref[...] == col).astype(jnp.float32)
    psum = lax.dot_general(onehot, h2, (((0,), (0,)), ((), ())),
                           preferred_element_type=jnp.float32)
    ones_col = jnp.ones((BN, 1), jnp.float32)
    csum = lax.dot_general(onehot, ones_col, (((0,), (0,)), ((), ())),
                           preferred_element_type=jnp.float32)

    @pl.when(i == 0)
    def _():
        acc_ref[...] = jnp.zeros_like(acc_ref)
        cnt_ref[...] = jnp.zeros_like(cnt_ref)

    acc_ref[...] += psum
    cnt_ref[...] += csum

    @pl.when(i == pl.num_programs(0) - 1)
    def _():
        out_ref[...] = acc_ref[...] / jnp.maximum(cnt_ref[...], 1.0)


def _final(p, invd, h, w, ws, gid2d):
    return pl.pallas_call(
        _final_body,
        grid=(N // BN,),
        in_specs=[
            pl.BlockSpec((NC, BN, H), lambda i: (0, i, 0)),
            pl.BlockSpec((BN, 1), lambda i: (i, 0)),
            pl.BlockSpec((BN, H), lambda i: (i, 0)),
            pl.BlockSpec((H, H), lambda i: (0, 0)),
            pl.BlockSpec((H, H), lambda i: (0, 0)),
            pl.BlockSpec((BN, 1), lambda i: (i, 0)),
        ],
        out_specs=pl.BlockSpec((G, H), lambda i: (0, 0)),
        out_shape=jax.ShapeDtypeStruct((G, H), jnp.float32),
        scratch_shapes=[
            pltpu.VMEM((G, H), jnp.float32),
            pltpu.VMEM((G, 1), jnp.float32),
        ],
    )(p, invd, h, w, ws, gid2d)


def kernel(node_types, edge_index, edge_types, graph_ids, node_emb, rel_emb,
           W, W_self):
    node_types = node_types.astype(jnp.int32)
    src = edge_index[0].astype(jnp.int32).reshape(NW, EPT)
    dst = edge_index[1].astype(jnp.int32).reshape(NW, EPT)
    typ = edge_types.astype(jnp.int32).reshape(NW, EPT)

    pad = PT - EPT
    pad_g = jnp.broadcast_to((jnp.arange(pad, dtype=jnp.int32) * 131) % N,
                             (NW, pad))
    pad_dst = jnp.broadcast_to(N + jnp.arange(pad, dtype=jnp.int32), (NW, pad))
    idx_all = jnp.stack([
        jnp.concatenate([typ * N + src, pad_g], axis=1).reshape(NW, CHUNKS, C),
        jnp.concatenate([dst, pad_dst], axis=1).reshape(NW, CHUNKS, C),
    ], axis=2)

    zeros = jnp.zeros((C, H), jnp.float32)
    ones = jnp.ones((C,), jnp.float32)
    emb_pad = jnp.pad(node_emb, ((0, 128 - node_emb.shape[0]), (0, 0)))

    h0, h4_0 = _featurize(node_types.reshape(N, 1), emb_pad, rel_emb[0])
    p0, degp = _edge_pass_deg()(h4_0.reshape(T_EDGE * N, H), idx_all,
                                zeros, ones)
    h1, h4_1, invd = _layer_update(p0, degp.reshape(NC, NPAD, 1), h0,
                                   W[0], W_self[0], rel_emb[1])
    (p1,) = _edge_pass()(h4_1.reshape(T_EDGE * N, H), idx_all, zeros, ones)
    pooled = _final(p1, invd, h1, W[1], W_self[1],
                    graph_ids.astype(jnp.int32).reshape(N, 1))
    return pooled


# speedup vs baseline: 1.1823x; 1.0856x over previous
import functools

import jax
import jax.numpy as jnp
from jax import lax
from jax.experimental import pallas as pl
from jax.experimental.pallas import tpu as pltpu
from jax.experimental.pallas import tpu_sc as plsc

N = 10000
E = 320000
H = 128
G = 512
T_EDGE = 4
NC = 2
NS = 16
NW = NC * NS
C = 64
EPT = E // NW
PT = 10240
CHUNKS = PT // C
NPAD = 10240
STRIPE = NPAD // NS
SCPT = STRIPE // C
BN = 2000


def _featurize_body(nt_ref, emb_ref, rel_ref, h_ref, h4_ref):
    col = lax.broadcasted_iota(jnp.int32, (BN, 128), 1)
    onehot = (nt_ref[...] == col).astype(jnp.float32)
    h = jnp.dot(onehot, emb_ref[...], preferred_element_type=jnp.float32)
    h_ref[...] = h
    for t in range(T_EDGE):
        h4_ref[t] = h * rel_ref[t]


def _featurize(nt2d, emb_pad, rel0):
    return pl.pallas_call(
        _featurize_body,
        grid=(N // BN,),
        in_specs=[
            pl.BlockSpec((BN, 1), lambda i: (i, 0)),
            pl.BlockSpec((128, H), lambda i: (0, 0)),
            pl.BlockSpec((T_EDGE, H), lambda i: (0, 0)),
        ],
        out_specs=[
            pl.BlockSpec((BN, H), lambda i: (i, 0)),
            pl.BlockSpec((T_EDGE, BN, H), lambda i: (0, i, 0)),
        ],
        out_shape=[
            jax.ShapeDtypeStruct((N, H), jnp.float32),
            jax.ShapeDtypeStruct((T_EDGE, N, H), jnp.float32),
        ],
    )(nt2d, emb_pad, rel0)


DEPTH = 4
UNROLL = 2
PAIRS = CHUNKS // (DEPTH * UNROLL)


def _make_edge_pass(with_deg):
    mesh = plsc.VectorSubcoreMesh(core_axis_name="c", subcore_axis_name="s",
                                  num_cores=NC, num_subcores=NS)
    out_type = [jax.ShapeDtypeStruct((NC, NPAD, H), jnp.float32)]
    if with_deg:
        out_type.append(jax.ShapeDtypeStruct((NC, NPAD), jnp.float32))
    NIB = DEPTH * UNROLL
    scratch = (
        [pltpu.VMEM((2, C), jnp.int32) for _ in range(NIB)]
        + [pltpu.VMEM((C, H), jnp.float32) for _ in range(DEPTH)]
        + [
            pltpu.VMEM((C,), jnp.float32),
            pltpu.VMEM_SHARED((NPAD, H), jnp.float32),
            pltpu.VMEM_SHARED((NPAD,), jnp.float32),
        ]
        + [pltpu.SemaphoreType.DMA for _ in range(NIB + 2 * DEPTH + 1)]
    )

    def body(h4_hbm, idx_hbm, zeros_hbm, ones_hbm, *rest):
        if with_deg:
            p_hbm, deg_hbm = rest[0], rest[1]
            scr = rest[2:]
        else:
            p_hbm = rest[0]
            deg_hbm = None
            scr = rest[1:]
        ib = scr[0:NIB]
        gb = scr[NIB:NIB + DEPTH]
        ones_v, agg_sh, deg_sh = scr[NIB + DEPTH:NIB + DEPTH + 3]
        isem = scr[NIB + DEPTH + 3:NIB + DEPTH + 3 + NIB]
        gsem = scr[NIB + DEPTH + 3 + NIB:NIB + DEPTH + 3 + NIB + DEPTH]
        ssem = scr[NIB + DEPTH + 3 + NIB + DEPTH:
                   NIB + DEPTH + 3 + NIB + 2 * DEPTH]
        dsem = scr[-1]
        cid = lax.axis_index("c")
        sid = lax.axis_index("s")
        w = cid * NS + sid
        base = sid * STRIPE

        pltpu.sync_copy(zeros_hbm, gb[0])
        pltpu.sync_copy(ones_hbm, ones_v)
        for k in range(SCPT):
            sl = pl.ds(base + k * C, C)
            pltpu.sync_copy(gb[0], agg_sh.at[sl])
            if with_deg:
                pltpu.sync_copy(gb[0].at[0, pl.ds(0, C)], deg_sh.at[sl])
        plsc.subcore_barrier()

        def pair_body(p, carry):
            c0 = p * NIB
            idesc = [pltpu.async_copy(idx_hbm.at[w, c0 + m], ib[m], isem[m])
                     for m in range(NIB)]
            prev_sd = None
            prev_dd = []
            for u in range(UNROLL):
                gd = []
                for j in range(DEPTH):
                    m = u * DEPTH + j
                    if prev_sd is not None:
                        prev_sd[j].wait()
                    idesc[m].wait()
                    gd.append(pltpu.async_copy(h4_hbm.at[ib[m].at[0]],
                                               gb[j], gsem[j]))
                for d in prev_dd:
                    d.wait()
                sd = []
                dd = []
                for j in range(DEPTH):
                    m = u * DEPTH + j
                    gd[j].wait()
                    sd.append(pltpu.async_copy(gb[j],
                                               agg_sh.at[ib[m].at[1]],
                                               ssem[j], add=True))
                    if with_deg:
                        dd.append(pltpu.async_copy(ones_v,
                                                   deg_sh.at[ib[m].at[1]],
                                                   dsem, add=True))
                prev_sd, prev_dd = sd, dd
            for d in prev_sd:
                d.wait()
            for d in prev_dd:
                d.wait()
            return carry

        lax.fori_loop(0, PAIRS, pair_body, 0)
        plsc.subcore_barrier()

        for k in range(SCPT):
            sl = pl.ds(base + k * C, C)
            pltpu.sync_copy(agg_sh.at[sl], gb[0])
            pltpu.sync_copy(gb[0], p_hbm.at[cid, sl])
            if with_deg:
                pltpu.sync_copy(deg_sh.at[sl], ones_v)
                pltpu.sync_copy(ones_v, deg_hbm.at[cid, sl])

    return functools.partial(
        pl.kernel, body, out_type=out_type, mesh=mesh, scratch_types=scratch
    )


@functools.lru_cache(maxsize=None)
def _edge_pass_deg():
    return _make_edge_pass(True)()


@functools.lru_cache(maxsize=None)
def _edge_pass():
    return _make_edge_pass(False)()


def _layer_body(p_ref, degp_ref, h_ref, w_ref, ws_ref, rel_ref,
                h1_ref, h4_ref, invd_ref):
    deg = jnp.maximum(degp_ref[0] + degp_ref[1], 1.0)
    invd = 1.0 / deg
    agg = (p_ref[0] + p_ref[1]) * invd
    h1 = jnp.maximum(
        jnp.dot(agg, w_ref[...], preferred_element_type=jnp.float32)
        + jnp.dot(h_ref[...], ws_ref[...], preferred_element_type=jnp.float32),
        0.0)
    h1_ref[...] = h1
    invd_ref[...] = invd
    for t in range(T_EDGE):
        h4_ref[t] = h1 * rel_ref[t]


def _layer_update(p, degp3, h, w, ws, rel1):
    return pl.pallas_call(
        _layer_body,
        grid=(N // BN,),
        in_specs=[
            pl.BlockSpec((NC, BN, H), lambda i: (0, i, 0)),
            pl.BlockSpec((NC, BN, 1), lambda i: (0, i, 0)),
            pl.BlockSpec((BN, H), lambda i: (i, 0)),
            pl.BlockSpec((H, H), lambda i: (0, 0)),
            pl.BlockSpec((H, H), lambda i: (0, 0)),
            pl.BlockSpec((T_EDGE, H), lambda i: (0, 0)),
        ],
        out_specs=[
            pl.BlockSpec((BN, H), lambda i: (i, 0)),
            pl.BlockSpec((T_EDGE, BN, H), lambda i: (0, i, 0)),
            pl.BlockSpec((BN, 1), lambda i: (i, 0)),
        ],
        out_shape=[
            jax.ShapeDtypeStruct((N, H), jnp.float32),
            jax.ShapeDtypeStruct((T_EDGE, N, H), jnp.float32),
            jax.ShapeDtypeStruct((N, 1), jnp.float32),
        ],
    )(p, degp3, h, w, ws, rel1)


def _final_body(p_ref, invd_ref, h_ref, w_ref, ws_ref, gid_ref,
                out_ref, acc_ref, cnt_ref):
    i = pl.program_id(0)
    agg = (p_ref[0] + p_ref[1]) * invd_ref[...]
    h2 = jnp.maximum(
        jnp.dot(agg, w_ref[...], preferred_element_type=jnp.float32)
        + jnp.dot(h_ref[...], ws_ref[...], preferred_element_type=jnp.float32),
        0.0)
    col = lax.broadcasted_iota(jnp.int32, (BN, G), 1)
    onehot = (gid_ref[...] == col).astype(jnp.float32)
    psum = lax.dot_general(onehot, h2, (((0,), (0,)), ((), ())),
                           preferred_element_type=jnp.float32)
    ones_col = jnp.ones((BN, 1), jnp.float32)
    csum = lax.dot_general(onehot, ones_col, (((0,), (0,)), ((), ())),
                           preferred_element_type=jnp.float32)

    @pl.when(i == 0)
    def _():
        acc_ref[...] = jnp.zeros_like(acc_ref)
        cnt_ref[...] = jnp.zeros_like(cnt_ref)

    acc_ref[...] += psum
    cnt_ref[...] += csum

    @pl.when(i == pl.num_programs(0) - 1)
    def _():
        out_ref[...] = acc_ref[...] / jnp.maximum(cnt_ref[...], 1.0)


def _final(p, invd, h, w, ws, gid2d):
    return pl.pallas_call(
        _final_body,
        grid=(N // BN,),
        in_specs=[
            pl.BlockSpec((NC, BN, H), lambda i: (0, i, 0)),
            pl.BlockSpec((BN, 1), lambda i: (i, 0)),
            pl.BlockSpec((BN, H), lambda i: (i, 0)),
            pl.BlockSpec((H, H), lambda i: (0, 0)),
            pl.BlockSpec((H, H), lambda i: (0, 0)),
            pl.BlockSpec((BN, 1), lambda i: (i, 0)),
        ],
        out_specs=pl.BlockSpec((G, H), lambda i: (0, 0)),
        out_shape=jax.ShapeDtypeStruct((G, H), jnp.float32),
        scratch_shapes=[
            pltpu.VMEM((G, H), jnp.float32),
            pltpu.VMEM((G, 1), jnp.float32),
        ],
    )(p, invd, h, w, ws, gid2d)


def kernel(node_types, edge_index, edge_types, graph_ids, node_emb, rel_emb,
           W, W_self):
    node_types = node_types.astype(jnp.int32)
    src = edge_index[0].astype(jnp.int32).reshape(NW, EPT)
    dst = edge_index[1].astype(jnp.int32).reshape(NW, EPT)
    typ = edge_types.astype(jnp.int32).reshape(NW, EPT)

    pad = PT - EPT
    pad_g = jnp.broadcast_to((jnp.arange(pad, dtype=jnp.int32) * 131) % N,
                             (NW, pad))
    pad_dst = jnp.broadcast_to(N + jnp.arange(pad, dtype=jnp.int32), (NW, pad))
    idx_all = jnp.stack([
        jnp.concatenate([typ * N + src, pad_g], axis=1).reshape(NW, CHUNKS, C),
        jnp.concatenate([dst, pad_dst], axis=1).reshape(NW, CHUNKS, C),
    ], axis=2)

    zeros = jnp.zeros((C, H), jnp.float32)
    ones = jnp.ones((C,), jnp.float32)
    emb_pad = jnp.pad(node_emb, ((0, 128 - node_emb.shape[0]), (0, 0)))

    h0, h4_0 = _featurize(node_types.reshape(N, 1), emb_pad, rel_emb[0])
    p0, degp = _edge_pass_deg()(h4_0.reshape(T_EDGE * N, H), idx_all,
                                zeros, ones)
    h1, h4_1, invd = _layer_update(p0, degp.reshape(NC, NPAD, 1), h0,
                                   W[0], W_self[0], rel_emb[1])
    (p1,) = _edge_pass()(h4_1.reshape(T_EDGE * N, H), idx_all, zeros, ones)
    pooled = _final(p1, invd, h1, W[1], W_self[1],
                    graph_ids.astype(jnp.int32).reshape(N, 1))
    return pooled
